# Initial kernel scaffold; baseline (speedup 1.0000x reference)
import jax
import jax.numpy as jnp
from jax.experimental import pallas as pl


def kernel(x, e_w1, e_b1, e_w2, e_b2, hop_w, n_w1, n_b1, n_w2, n_b2, n_w3, n_b3):
    raise NotImplementedError("write your pallas kernel here")


# speedup vs baseline: 5.6858x; 5.6858x over previous
import functools

import jax
import jax.numpy as jnp
from jax import lax
from jax.experimental import pallas as pl
from jax.experimental.pallas import tpu as pltpu

_E = 64
_N_ITER = 10
_BM = 512


def _body(x_ref, ew1_ref, eb1_ref, ew2_ref, eb2t_ref, hopbf_ref, wm_ref,
          nw1x_ref, nw1r_ref, nb1_ref, nw2_ref, nb2_ref, nw3_ref, nb3_ref,
          out_ref, *, bm):
    f32 = jnp.float32
    xb = x_ref[...]
    h1 = jax.nn.relu(jnp.dot(xb, ew1_ref[...],
                             preferred_element_type=f32) + eb1_ref[...])
    enc_t = lax.dot_general(ew2_ref[...], h1, (((0,), (1,)), ((), ())),
                            preferred_element_type=f32)
    enc_t = enc_t + pltpu.repeat(eb2t_ref[...], bm // 128, axis=1)
    state_t = jnp.where(enc_t > 0, 1.0, -1.0)

    act_t = jnp.dot(hopbf_ref[...], state_t.astype(jnp.bfloat16),
                    preferred_element_type=f32)

    s = jnp.concatenate([act_t, state_t], axis=0)

    def sweep(_, s):
        for i in range(_E):
            a = s[i:i + 1, :]
            old = s[_E + i:_E + i + 1, :]
            new = jnp.where(a > 0, 1.0, -1.0)
            d = new - old
            u = pltpu.repeat(wm_ref[i], bm // 128,
                             axis=1).astype(jnp.float32)
            s = s + u * d
        return s

    s = lax.fori_loop(0, _N_ITER, sweep, s)
    retr_t = jnp.where(s[_E:, :] > 0, 1.0, 0.0)

    h = jax.nn.relu(
        jnp.dot(xb, nw1x_ref[...], preferred_element_type=f32)
        + lax.dot_general(retr_t, nw1r_ref[...], (((0,), (0,)), ((), ())),
                          preferred_element_type=f32)
        + nb1_ref[...])
    h = jax.nn.relu(jnp.dot(h, nw2_ref[...],
                            preferred_element_type=f32) + nb2_ref[...])
    out_ref[...] = jnp.dot(h, nw3_ref[...],
                           preferred_element_type=f32) + nb3_ref[...]


def kernel(x, e_w1, e_b1, e_w2, e_b2, hop_w, n_w1, n_b1, n_w2, n_b2, n_w3,
           n_b3, *, interpret=False):
    b, in_dim = x.shape
    hid = e_w1.shape[1]
    out_dim = n_w3.shape[1]
    bm = _BM if b % _BM == 0 else b
    nb = b // bm

    hop_bf = hop_w.astype(jnp.bfloat16)
    wm2 = jnp.concatenate([hop_w.T, jnp.eye(_E, dtype=jnp.float32)],
                          axis=1)
    wm = jnp.broadcast_to(wm2[:, :, None],
                          (_E, 2 * _E, 128)).astype(jnp.bfloat16)
    eb2t = jnp.broadcast_to(e_b2[:, None], (_E, 128))

    n_w1x = n_w1[:in_dim]
    n_w1r = n_w1[in_dim:]

    const = lambda *bs: pl.BlockSpec(bs, lambda i: tuple(0 for _ in bs))
    grid_spec = pl.GridSpec(
        grid=(nb,),
        in_specs=[
            pl.BlockSpec((bm, in_dim), lambda i: (i, 0)),
            const(in_dim, hid),
            const(1, hid),
            const(hid, _E),
            const(_E, 128),
            const(_E, _E),
            const(_E, 2 * _E, 128),
            const(in_dim, hid),
            const(_E, hid),
            const(1, hid),
            const(hid, hid),
            const(1, hid),
            const(hid, out_dim),
            const(1, out_dim),
        ],
        out_specs=pl.BlockSpec((bm, out_dim), lambda i: (i, 0)),
    )
    return pl.pallas_call(
        functools.partial(_body, bm=bm),
        grid_spec=grid_spec,
        out_shape=jax.ShapeDtypeStruct((b, out_dim), jnp.float32),
        compiler_params=pltpu.CompilerParams(
            dimension_semantics=("arbitrary",),
        ),
        name="hopfield_dqn",
        interpret=interpret,
    )(x, e_w1, e_b1[None, :], e_w2, eb2t, hop_bf, wm,
      n_w1x, n_w1r, n_b1[None, :], n_w2, n_b2[None, :], n_w3, n_b3[None, :])


# speedup vs baseline: 9.9984x; 1.7585x over previous
import functools

import jax
import jax.numpy as jnp
from jax import lax
from jax.experimental import pallas as pl
from jax.experimental.pallas import tpu as pltpu

_E = 64
_N_ITER = 10
_BM = 512


def _body(x_ref, ew1_ref, eb1_ref, ew2_ref, eb2t_ref, hopbf_ref, wb_ref,
          nw1x_ref, nw1r_ref, nb1_ref, nw2_ref, nb2_ref, nw3_ref, nb3_ref,
          out_ref, state_ref, *, bm):
    f32 = jnp.float32
    bf16 = jnp.bfloat16
    xb = x_ref[...]
    h1 = jax.nn.relu(jnp.dot(xb, ew1_ref[...],
                             preferred_element_type=f32) + eb1_ref[...])
    enc_t = lax.dot_general(ew2_ref[...], h1.astype(bf16),
                            (((0,), (1,)), ((), ())),
                            preferred_element_type=f32)
    enc_t = enc_t + pltpu.repeat(eb2t_ref[...], bm // 128, axis=1)
    state_t = jnp.where(enc_t > 0, 1.0, -1.0)
    state_ref[...] = state_t

    act_t = jnp.dot(hopbf_ref[...], state_t.astype(bf16),
                    preferred_element_type=f32)

    def sweep(_, act):
        for i in range(_E):
            a = act[i:i + 1, :]
            old = state_ref[i:i + 1, :]
            new = jnp.where(a > 0, 1.0, -1.0)
            d = new - old
            state_ref[i:i + 1, :] = new
            wcol = pltpu.repeat(wb_ref[i], bm // 128,
                                axis=1).astype(f32)
            act = act + wcol * d
        return act

    lax.fori_loop(0, _N_ITER, sweep, act_t)
    retr_t = jnp.where(state_ref[...] > 0, 1.0, 0.0)

    h = jax.nn.relu(
        jnp.dot(xb, nw1x_ref[...], preferred_element_type=f32)
        + lax.dot_general(retr_t.astype(bf16), nw1r_ref[...],
                          (((0,), (0,)), ((), ())),
                          preferred_element_type=f32)
        + nb1_ref[...])
    h = jax.nn.relu(jnp.dot(h.astype(bf16), nw2_ref[...],
                            preferred_element_type=f32) + nb2_ref[...])
    out_ref[...] = jnp.dot(h.astype(bf16), nw3_ref[...],
                           preferred_element_type=f32) + nb3_ref[...]


def kernel(x, e_w1, e_b1, e_w2, e_b2, hop_w, n_w1, n_b1, n_w2, n_b2, n_w3,
           n_b3, *, interpret=False):
    b, in_dim = x.shape
    hid = e_w1.shape[1]
    out_dim = n_w3.shape[1]
    bm = _BM if b % _BM == 0 else b
    nb = b // bm
    bf16 = jnp.bfloat16

    hop_bf = hop_w.astype(bf16)
    wb = jnp.broadcast_to(hop_w.T[:, :, None], (_E, _E, 128)).astype(bf16)
    eb2t = jnp.broadcast_to(e_b2[:, None], (_E, 128))

    n_w1x = n_w1[:in_dim]
    n_w1r = n_w1[in_dim:]

    const = lambda *bs: pl.BlockSpec(bs, lambda i: tuple(0 for _ in bs))
    return pl.pallas_call(
        functools.partial(_body, bm=bm),
        grid=(nb,),
        in_specs=[
            pl.BlockSpec((bm, in_dim), lambda i: (i, 0)),
            const(in_dim, hid),
            const(1, hid),
            const(hid, _E),
            const(_E, 128),
            const(_E, _E),
            const(_E, _E, 128),
            const(in_dim, hid),
            const(_E, hid),
            const(1, hid),
            const(hid, hid),
            const(1, hid),
            const(hid, out_dim),
            const(1, out_dim),
        ],
        out_specs=pl.BlockSpec((bm, out_dim), lambda i: (i, 0)),
        out_shape=jax.ShapeDtypeStruct((b, out_dim), jnp.float32),
        scratch_shapes=[pltpu.VMEM((_E, bm), jnp.float32)],
        compiler_params=pltpu.CompilerParams(
            dimension_semantics=("arbitrary",),
        ),
        name="hopfield_dqn",
        interpret=interpret,
    )(x.astype(bf16), e_w1.astype(bf16), e_b1[None, :], e_w2.astype(bf16),
      eb2t, hop_bf, wb, n_w1x.astype(bf16), n_w1r.astype(bf16),
      n_b1[None, :], n_w2.astype(bf16), n_b2[None, :], n_w3.astype(bf16),
      n_b3[None, :])
